# Initial kernel scaffold; baseline (speedup 1.0000x reference)
import jax
import jax.numpy as jnp
from jax.experimental import pallas as pl


def kernel(x, edge_index, batch, W1, W2, Wfc):
    raise NotImplementedError("write your pallas kernel here")


# speedup vs baseline: 9.6853x; 9.6853x over previous
import functools

import jax
import jax.numpy as jnp
from jax import lax
from jax.experimental import pallas as pl
from jax.experimental.pallas import tpu as pltpu
from jax.experimental.pallas import tpu_sc as plsc

F32 = jnp.float32
_N = 10000
_E = 320000
_G = 64
_W = 16
_CH = 128
_NCH = _E // _CH
_NC, _NS = 2, 16
_NWORK = _NC * _NS
_CPW = _NCH // _NWORK
_ZRA = 624
_ZRL = _N - (_NS - 1) * _ZRA
_RB = 1000
_NB = _N // _RB


def _segsum_body(tab_hbm, src_hbm, dst_hbm, zeros_hbm, out_hbm,
                 sidx, didx, rows, acc, sem):
    c = lax.axis_index("c")
    s = lax.axis_index("s")
    wid = c * _NS + s
    rows_a = pl.ds(s * _ZRA, _ZRA)
    rows_l = pl.ds((_NS - 1) * _ZRA, _ZRL)

    @pl.when(s < _NS - 1)
    def _():
        pltpu.sync_copy(zeros_hbm.at[rows_a], acc.at[rows_a])

    @pl.when(s == _NS - 1)
    def _():
        pltpu.sync_copy(zeros_hbm.at[rows_l], acc.at[rows_l])

    plsc.subcore_barrier()

    def do_chunk(ci):
        base = ci * _CH
        pltpu.sync_copy(src_hbm.at[pl.ds(base, _CH)], sidx)
        pltpu.sync_copy(dst_hbm.at[pl.ds(base, _CH)], didx)
        pltpu.async_copy(tab_hbm.at[sidx], rows, sem).wait()
        pltpu.sync_copy(rows, acc.at[didx], add=True)

    def body(i, carry):
        do_chunk(wid * _CPW + i)
        return carry

    lax.fori_loop(0, _CPW, body, 0)

    @pl.when(wid < _NCH - _CPW * _NWORK)
    def _():
        do_chunk(_CPW * _NWORK + wid)

    plsc.subcore_barrier()

    @pl.when(s < _NS - 1)
    def _():
        pltpu.sync_copy(acc.at[rows_a], out_hbm.at[c, rows_a])

    @pl.when(s == _NS - 1)
    def _():
        pltpu.sync_copy(acc.at[rows_l], out_hbm.at[c, rows_l])


@functools.cache
def _make_segsum():
    return pl.kernel(
        _segsum_body,
        out_type=jax.ShapeDtypeStruct((_NC, _N, _W), F32),
        mesh=plsc.VectorSubcoreMesh(
            core_axis_name="c", subcore_axis_name="s",
            num_cores=_NC, num_subcores=_NS),
        scratch_types=[
            pltpu.VMEM((_CH,), jnp.int32),
            pltpu.VMEM((_CH,), jnp.int32),
            pltpu.VMEM((_CH, _W), F32),
            pltpu.VMEM_SHARED((_N, _W), F32),
            pltpu.SemaphoreType.DMA,
        ],
        compiler_params=pltpu.CompilerParams(use_tc_tiling_on_sc=False),
    )


def _k1_body(x_ref, w_ref, self_ref, tab_ref):
    y = lax.dot_general(x_ref[...], w_ref[...], (((1,), (0,)), ((), ())),
                        precision=lax.Precision.HIGHEST,
                        preferred_element_type=F32)
    cols = lax.broadcasted_iota(jnp.int32, (_RB, 2 * _W), 1)
    y = y + jnp.where(cols == _W + 10, 1.0, 0.0).astype(F32)
    self_ref[...] = y[:, :_W]
    tab_ref[...] = y[:, _W:]


def _k3_body(self_ref, p_ref, w_ref, self2_ref, tab2_ref):
    agg = p_ref[0] + p_ref[1]
    dinv = 1.0 / jnp.maximum(agg[:, 10:11], 1.0)
    h = jnp.maximum(self_ref[...] + agg * dinv, 0.0)
    y = lax.dot_general(h, w_ref[...], (((1,), (0,)), ((), ())),
                        precision=lax.Precision.HIGHEST,
                        preferred_element_type=F32)
    cols = lax.broadcasted_iota(jnp.int32, (_RB, 2 * _W), 1)
    y = y + jnp.where(cols == _W + 10, 1.0, 0.0).astype(F32)
    self2_ref[...] = y[:, :_W]
    tab2_ref[...] = y[:, _W:]


def _k5_body(self_ref, p_ref, b_ref, wfc_ref, out_ref, acc):
    i = pl.program_id(0)
    agg = p_ref[0] + p_ref[1]
    dinv = 1.0 / jnp.maximum(agg[:, 10:11], 1.0)
    h2 = self_ref[...] + agg * dinv
    cols = lax.broadcasted_iota(jnp.int32, (_RB, _W), 1)
    h2 = jnp.where(cols == 10, 1.0, h2)
    rows = lax.broadcasted_iota(jnp.int32, (_G, _RB), 0)
    onehot = (rows == b_ref[0]).astype(F32)
    part = lax.dot_general(onehot, h2, (((1,), (0,)), ((), ())),
                           precision=lax.Precision.HIGHEST,
                           preferred_element_type=F32)

    @pl.when(i == 0)
    def _():
        acc[...] = jnp.zeros((_G, _W), F32)

    acc[...] += part

    @pl.when(i == _NB - 1)
    def _():
        pooled = acc[...] / jnp.maximum(acc[:, 10:11], 1.0)
        out_ref[...] = jax.nn.sigmoid(
            lax.dot_general(pooled, wfc_ref[...], (((1,), (0,)), ((), ())),
                            precision=lax.Precision.HIGHEST,
                            preferred_element_type=F32))


_k1 = pl.pallas_call(
    _k1_body,
    grid=(_NB,),
    in_specs=[
        pl.BlockSpec((_RB, 128), lambda i: (i, 0)),
        pl.BlockSpec((128, 2 * _W), lambda i: (0, 0)),
    ],
    out_specs=[
        pl.BlockSpec((_RB, _W), lambda i: (i, 0)),
        pl.BlockSpec((_RB, _W), lambda i: (i, 0)),
    ],
    out_shape=[jax.ShapeDtypeStruct((_N, _W), F32)] * 2,
)

_k3 = pl.pallas_call(
    _k3_body,
    grid=(_NB,),
    in_specs=[
        pl.BlockSpec((_RB, _W), lambda i: (i, 0)),
        pl.BlockSpec((_NC, _RB, _W), lambda i: (0, i, 0)),
        pl.BlockSpec((_W, 2 * _W), lambda i: (0, 0)),
    ],
    out_specs=[
        pl.BlockSpec((_RB, _W), lambda i: (i, 0)),
        pl.BlockSpec((_RB, _W), lambda i: (i, 0)),
    ],
    out_shape=[jax.ShapeDtypeStruct((_N, _W), F32)] * 2,
)

_k5 = pl.pallas_call(
    _k5_body,
    grid=(_NB,),
    in_specs=[
        pl.BlockSpec((_RB, _W), lambda i: (i, 0)),
        pl.BlockSpec((_NC, _RB, _W), lambda i: (0, i, 0)),
        pl.BlockSpec((1, 1, _RB), lambda i: (i, 0, 0)),
        pl.BlockSpec((_W, 1), lambda i: (0, 0)),
    ],
    out_specs=pl.BlockSpec((_G, 1), lambda i: (0, 0)),
    out_shape=jax.ShapeDtypeStruct((_G, 1), F32),
    scratch_shapes=[pltpu.VMEM((_G, _W), F32)],
)


def _pack_w1(W1):
    w = jnp.zeros((128, 2 * _W), F32)
    w = w.at[:, :10].set(W1[:128])
    return w.at[:, _W:_W + 10].set(W1[128:])


def _pack_w2(W2):
    w = jnp.zeros((_W, 2 * _W), F32)
    w = w.at[:10, :10].set(W2[:10])
    return w.at[:10, _W:_W + 10].set(W2[10:])


def kernel(x, edge_index, batch, W1, W2, Wfc):
    src = edge_index[0]
    dst = edge_index[1]
    zeros = jnp.zeros((_N, _W), F32)
    wfc = jnp.zeros((_W, 1), F32).at[:10].set(Wfc)
    batch3 = batch.reshape(_NB, 1, _RB)

    segsum = _make_segsum()
    self1, tab1 = _k1(x, _pack_w1(W1))
    p1 = segsum(tab1, src, dst, zeros)
    self2, tab2 = _k3(self1, p1, _pack_w2(W2))
    p2 = segsum(tab2, src, dst, zeros)
    return _k5(self2, p2, batch3, wfc)


# speedup vs baseline: 25.3223x; 2.6145x over previous
import functools

import jax
import jax.numpy as jnp
from jax import lax
from jax.experimental import pallas as pl
from jax.experimental.pallas import tpu as pltpu
from jax.experimental.pallas import tpu_sc as plsc

F32 = jnp.float32
_N = 10000
_E = 320000
_G = 64
_W = 16
_CH = 128
_NCH = _E // _CH
_NC, _NS = 2, 16
_NWORK = _NC * _NS
_CPW = _NCH // _NWORK
_ZRA = 624
_ZRL = _N - (_NS - 1) * _ZRA
_RB = 1000
_NB = _N // _RB


_GRP = 6
_NGRP = _CPW // _GRP


def _segsum_body(tab_hbm, src_hbm, dst_hbm, zeros_hbm, out_hbm,
                 sidx, didx, rows, acc, sem0, sem1):
    c = lax.axis_index("c")
    s = lax.axis_index("s")
    wid = c * _NS + s
    rows_a = pl.ds(s * _ZRA, _ZRA)
    rows_l = pl.ds((_NS - 1) * _ZRA, _ZRL)
    sems = (sem0, sem1)

    my_chunks = pl.ds(wid * _CPW, _CPW)
    pltpu.sync_copy(src_hbm.at[my_chunks], sidx.at[pl.ds(0, _CPW)])
    pltpu.sync_copy(dst_hbm.at[my_chunks], didx.at[pl.ds(0, _CPW)])
    has_extra = wid < _NCH - _CPW * _NWORK
    extra = pl.ds(_CPW * _NWORK + wid, 1)

    @pl.when(has_extra)
    def _():
        pltpu.sync_copy(src_hbm.at[extra], sidx.at[pl.ds(_CPW, 1)])
        pltpu.sync_copy(dst_hbm.at[extra], didx.at[pl.ds(_CPW, 1)])

    @pl.when(s < _NS - 1)
    def _():
        pltpu.sync_copy(zeros_hbm.at[rows_a], acc.at[rows_a])

    @pl.when(s == _NS - 1)
    def _():
        pltpu.sync_copy(zeros_hbm.at[rows_l], acc.at[rows_l])

    plsc.subcore_barrier()

    def fire(g, p):
        for j in range(_GRP):
            pltpu.async_copy(tab_hbm.at[sidx.at[g * _GRP + j]],
                             rows.at[p, j], sems[p])

    def drain_scatter(g, p):
        for j in range(_GRP):
            pltpu.make_async_copy(zeros_hbm.at[pl.ds(0, _CH)],
                                  rows.at[p, j], sems[p]).wait()
        for j in range(_GRP):
            pltpu.sync_copy(rows.at[p, j], acc.at[didx.at[g * _GRP + j]],
                            add=True)

    fire(0, 0)

    def body(g, carry):
        p = lax.rem(g, 2)

        @pl.when(p == 0)
        def _():
            fire(g, 0)
            drain_scatter(g - 1, 1)

        @pl.when(p == 1)
        def _():
            fire(g, 1)
            drain_scatter(g - 1, 0)

        return carry

    lax.fori_loop(1, _NGRP, body, 0)
    drain_scatter(_NGRP - 1, (_NGRP - 1) % 2)

    @pl.when(has_extra)
    def _():
        pltpu.async_copy(tab_hbm.at[sidx.at[_CPW]], rows.at[0, 0], sem0)
        pltpu.make_async_copy(zeros_hbm.at[pl.ds(0, _CH)],
                              rows.at[0, 0], sem0).wait()
        pltpu.sync_copy(rows.at[0, 0], acc.at[didx.at[_CPW]], add=True)

    plsc.subcore_barrier()

    @pl.when(s < _NS - 1)
    def _():
        pltpu.sync_copy(acc.at[rows_a], out_hbm.at[c, rows_a])

    @pl.when(s == _NS - 1)
    def _():
        pltpu.sync_copy(acc.at[rows_l], out_hbm.at[c, rows_l])


@functools.cache
def _make_segsum():
    return pl.kernel(
        _segsum_body,
        out_type=jax.ShapeDtypeStruct((_NC, _N, _W), F32),
        mesh=plsc.VectorSubcoreMesh(
            core_axis_name="c", subcore_axis_name="s",
            num_cores=_NC, num_subcores=_NS),
        scratch_types=[
            pltpu.VMEM((_CPW + 1, _CH), jnp.int32),
            pltpu.VMEM((_CPW + 1, _CH), jnp.int32),
            pltpu.VMEM((2, _GRP, _CH, _W), F32),
            pltpu.VMEM_SHARED((_N, _W), F32),
            pltpu.SemaphoreType.DMA,
            pltpu.SemaphoreType.DMA,
        ],
        compiler_params=pltpu.CompilerParams(use_tc_tiling_on_sc=False),
    )


def _k1_body(x_ref, w_ref, self_ref, tab_ref):
    y = lax.dot_general(x_ref[...], w_ref[...], (((1,), (0,)), ((), ())),
                        precision=lax.Precision.HIGHEST,
                        preferred_element_type=F32)
    cols = lax.broadcasted_iota(jnp.int32, (_RB, 2 * _W), 1)
    y = y + jnp.where(cols == _W + 10, 1.0, 0.0).astype(F32)
    self_ref[...] = y[:, :_W]
    tab_ref[...] = y[:, _W:]


def _k3_body(self_ref, p_ref, w_ref, self2_ref, tab2_ref):
    agg = p_ref[0] + p_ref[1]
    dinv = 1.0 / jnp.maximum(agg[:, 10:11], 1.0)
    h = jnp.maximum(self_ref[...] + agg * dinv, 0.0)
    y = lax.dot_general(h, w_ref[...], (((1,), (0,)), ((), ())),
                        precision=lax.Precision.HIGHEST,
                        preferred_element_type=F32)
    cols = lax.broadcasted_iota(jnp.int32, (_RB, 2 * _W), 1)
    y = y + jnp.where(cols == _W + 10, 1.0, 0.0).astype(F32)
    self2_ref[...] = y[:, :_W]
    tab2_ref[...] = y[:, _W:]


def _k5_body(self_ref, p_ref, b_ref, wfc_ref, out_ref, acc):
    i = pl.program_id(0)
    agg = p_ref[0] + p_ref[1]
    dinv = 1.0 / jnp.maximum(agg[:, 10:11], 1.0)
    h2 = self_ref[...] + agg * dinv
    cols = lax.broadcasted_iota(jnp.int32, (_RB, _W), 1)
    h2 = jnp.where(cols == 10, 1.0, h2)
    rows = lax.broadcasted_iota(jnp.int32, (_G, _RB), 0)
    onehot = (rows == b_ref[0]).astype(F32)
    part = lax.dot_general(onehot, h2, (((1,), (0,)), ((), ())),
                           precision=lax.Precision.HIGHEST,
                           preferred_element_type=F32)

    @pl.when(i == 0)
    def _():
        acc[...] = jnp.zeros((_G, _W), F32)

    acc[...] += part

    @pl.when(i == _NB - 1)
    def _():
        pooled = acc[...] / jnp.maximum(acc[:, 10:11], 1.0)
        out_ref[...] = jax.nn.sigmoid(
            lax.dot_general(pooled, wfc_ref[...], (((1,), (0,)), ((), ())),
                            precision=lax.Precision.HIGHEST,
                            preferred_element_type=F32))


_k1 = pl.pallas_call(
    _k1_body,
    grid=(_NB,),
    in_specs=[
        pl.BlockSpec((_RB, 128), lambda i: (i, 0)),
        pl.BlockSpec((128, 2 * _W), lambda i: (0, 0)),
    ],
    out_specs=[
        pl.BlockSpec((_RB, _W), lambda i: (i, 0)),
        pl.BlockSpec((_RB, _W), lambda i: (i, 0)),
    ],
    out_shape=[jax.ShapeDtypeStruct((_N, _W), F32)] * 2,
)

_k3 = pl.pallas_call(
    _k3_body,
    grid=(_NB,),
    in_specs=[
        pl.BlockSpec((_RB, _W), lambda i: (i, 0)),
        pl.BlockSpec((_NC, _RB, _W), lambda i: (0, i, 0)),
        pl.BlockSpec((_W, 2 * _W), lambda i: (0, 0)),
    ],
    out_specs=[
        pl.BlockSpec((_RB, _W), lambda i: (i, 0)),
        pl.BlockSpec((_RB, _W), lambda i: (i, 0)),
    ],
    out_shape=[jax.ShapeDtypeStruct((_N, _W), F32)] * 2,
)

_k5 = pl.pallas_call(
    _k5_body,
    grid=(_NB,),
    in_specs=[
        pl.BlockSpec((_RB, _W), lambda i: (i, 0)),
        pl.BlockSpec((_NC, _RB, _W), lambda i: (0, i, 0)),
        pl.BlockSpec((1, 1, _RB), lambda i: (i, 0, 0)),
        pl.BlockSpec((_W, 1), lambda i: (0, 0)),
    ],
    out_specs=pl.BlockSpec((_G, 1), lambda i: (0, 0)),
    out_shape=jax.ShapeDtypeStruct((_G, 1), F32),
    scratch_shapes=[pltpu.VMEM((_G, _W), F32)],
)


def _pack_w1(W1):
    w = jnp.zeros((128, 2 * _W), F32)
    w = w.at[:, :10].set(W1[:128])
    return w.at[:, _W:_W + 10].set(W1[128:])


def _pack_w2(W2):
    w = jnp.zeros((_W, 2 * _W), F32)
    w = w.at[:10, :10].set(W2[:10])
    return w.at[:10, _W:_W + 10].set(W2[10:])


def kernel(x, edge_index, batch, W1, W2, Wfc):
    src = edge_index[0].reshape(_NCH, _CH)
    dst = edge_index[1].reshape(_NCH, _CH)
    zeros = jnp.zeros((_N, _W), F32)
    wfc = jnp.zeros((_W, 1), F32).at[:10].set(Wfc)
    batch3 = batch.reshape(_NB, 1, _RB)

    segsum = _make_segsum()
    self1, tab1 = _k1(x, _pack_w1(W1))
    p1 = segsum(tab1, src, dst, zeros)
    self2, tab2 = _k3(self1, p1, _pack_w2(W2))
    p2 = segsum(tab2, src, dst, zeros)
    return _k5(self2, p2, batch3, wfc)


# speedup vs baseline: 25.5932x; 1.0107x over previous
import functools

import jax
import jax.numpy as jnp
from jax import lax
from jax.experimental import pallas as pl
from jax.experimental.pallas import tpu as pltpu
from jax.experimental.pallas import tpu_sc as plsc

F32 = jnp.float32
_N = 10000
_E = 320000
_G = 64
_W = 16
_CH = 128
_NCH = _E // _CH
_NC, _NS = 2, 16
_NWORK = _NC * _NS
_CPW = _NCH // _NWORK
_ZRA = 624
_ZRL = _N - (_NS - 1) * _ZRA
_RB = 1000
_NB = _N // _RB


_GRP = 6
_NGRP = _CPW // _GRP


def _segsum_body(tab_hbm, edges_hbm, zeros_hbm, out_hbm,
                 sidx, didx, rows, acc, sem0, sem1):
    c = lax.axis_index("c")
    s = lax.axis_index("s")
    wid = c * _NS + s
    rows_a = pl.ds(s * _ZRA, _ZRA)
    rows_l = pl.ds((_NS - 1) * _ZRA, _ZRL)
    sems = (sem0, sem1)

    my_chunks = pl.ds(wid * _CPW, _CPW)
    pltpu.sync_copy(edges_hbm.at[0, my_chunks], sidx.at[pl.ds(0, _CPW)])
    pltpu.sync_copy(edges_hbm.at[1, my_chunks], didx.at[pl.ds(0, _CPW)])
    has_extra = wid < _NCH - _CPW * _NWORK
    extra = pl.ds(_CPW * _NWORK + wid, 1)

    @pl.when(has_extra)
    def _():
        pltpu.sync_copy(edges_hbm.at[0, extra], sidx.at[pl.ds(_CPW, 1)])
        pltpu.sync_copy(edges_hbm.at[1, extra], didx.at[pl.ds(_CPW, 1)])

    @pl.when(s < _NS - 1)
    def _():
        pltpu.sync_copy(zeros_hbm.at[rows_a], acc.at[rows_a])

    @pl.when(s == _NS - 1)
    def _():
        pltpu.sync_copy(zeros_hbm.at[rows_l], acc.at[rows_l])

    plsc.subcore_barrier()

    def fire(g, p):
        for j in range(_GRP):
            pltpu.async_copy(tab_hbm.at[sidx.at[g * _GRP + j]],
                             rows.at[p, j], sems[p])

    def drain_scatter(g, p):
        for j in range(_GRP):
            pltpu.make_async_copy(zeros_hbm.at[pl.ds(0, _CH)],
                                  rows.at[p, j], sems[p]).wait()
        for j in range(_GRP):
            pltpu.sync_copy(rows.at[p, j], acc.at[didx.at[g * _GRP + j]],
                            add=True)

    fire(0, 0)

    def body(g, carry):
        p = lax.rem(g, 2)

        @pl.when(p == 0)
        def _():
            fire(g, 0)
            drain_scatter(g - 1, 1)

        @pl.when(p == 1)
        def _():
            fire(g, 1)
            drain_scatter(g - 1, 0)

        return carry

    lax.fori_loop(1, _NGRP, body, 0)
    drain_scatter(_NGRP - 1, (_NGRP - 1) % 2)

    @pl.when(has_extra)
    def _():
        pltpu.async_copy(tab_hbm.at[sidx.at[_CPW]], rows.at[0, 0], sem0)
        pltpu.make_async_copy(zeros_hbm.at[pl.ds(0, _CH)],
                              rows.at[0, 0], sem0).wait()
        pltpu.sync_copy(rows.at[0, 0], acc.at[didx.at[_CPW]], add=True)

    plsc.subcore_barrier()

    @pl.when(s < _NS - 1)
    def _():
        pltpu.sync_copy(acc.at[rows_a], out_hbm.at[c, rows_a])

    @pl.when(s == _NS - 1)
    def _():
        pltpu.sync_copy(acc.at[rows_l], out_hbm.at[c, rows_l])


@functools.cache
def _make_segsum():
    return pl.kernel(
        _segsum_body,
        out_type=jax.ShapeDtypeStruct((_NC, _N, _W), F32),
        mesh=plsc.VectorSubcoreMesh(
            core_axis_name="c", subcore_axis_name="s",
            num_cores=_NC, num_subcores=_NS),
        scratch_types=[
            pltpu.VMEM((_CPW + 1, _CH), jnp.int32),
            pltpu.VMEM((_CPW + 1, _CH), jnp.int32),
            pltpu.VMEM((2, _GRP, _CH, _W), F32),
            pltpu.VMEM_SHARED((_N, _W), F32),
            pltpu.SemaphoreType.DMA,
            pltpu.SemaphoreType.DMA,
        ],
        compiler_params=pltpu.CompilerParams(use_tc_tiling_on_sc=False),
    )


def _halves(y_top, y_bot):
    z5 = jnp.zeros((_RB, _W - 11), F32)
    z1 = jnp.zeros((_RB, 1), F32)
    ones = jnp.ones((_RB, 1), F32)
    self_half = jnp.concatenate([y_top, z1, z5], axis=1)
    tab_half = jnp.concatenate([y_bot, ones, z5], axis=1)
    return self_half, tab_half


def _dot(a, b):
    return lax.dot_general(a, b, (((1,), (0,)), ((), ())),
                           precision=lax.Precision.HIGHEST,
                           preferred_element_type=F32)


def _k1_body(x_ref, w_ref, self_ref, tab_ref):
    x = x_ref[...]
    self_ref[...], tab_ref[...] = _halves(_dot(x, w_ref[:128]),
                                          _dot(x, w_ref[128:]))


def _k3_body(self_ref, p_ref, w_ref, self2_ref, tab2_ref):
    agg = p_ref[0] + p_ref[1]
    dinv = 1.0 / jnp.maximum(agg[:, 10:11], 1.0)
    h = jnp.maximum((self_ref[...] + agg * dinv)[:, :10], 0.0)
    self2_ref[...], tab2_ref[...] = _halves(_dot(h, w_ref[:10]),
                                            _dot(h, w_ref[10:]))


def _k5_body(self_ref, p_ref, b_ref, wfc_ref, out_ref, acc):
    i = pl.program_id(0)
    agg = p_ref[0] + p_ref[1]
    dinv = 1.0 / jnp.maximum(agg[:, 10:11], 1.0)
    h2 = self_ref[...] + agg * dinv
    cols = lax.broadcasted_iota(jnp.int32, (_RB, _W), 1)
    h2 = jnp.where(cols == 10, 1.0, h2)
    rows = lax.broadcasted_iota(jnp.int32, (_G, _RB), 0)
    onehot = (rows == b_ref[0]).astype(F32)
    part = lax.dot_general(onehot, h2, (((1,), (0,)), ((), ())),
                           precision=lax.Precision.HIGHEST,
                           preferred_element_type=F32)

    @pl.when(i == 0)
    def _():
        acc[...] = jnp.zeros((_G, _W), F32)

    acc[...] += part

    @pl.when(i == _NB - 1)
    def _():
        pooled = (acc[...] / jnp.maximum(acc[:, 10:11], 1.0))[:, :10]
        out_ref[...] = jax.nn.sigmoid(_dot(pooled, wfc_ref[...]))


_k1 = pl.pallas_call(
    _k1_body,
    grid=(_NB,),
    in_specs=[
        pl.BlockSpec((_RB, 128), lambda i: (i, 0)),
        pl.BlockSpec((256, 10), lambda i: (0, 0)),
    ],
    out_specs=[
        pl.BlockSpec((_RB, _W), lambda i: (i, 0)),
        pl.BlockSpec((_RB, _W), lambda i: (i, 0)),
    ],
    out_shape=[jax.ShapeDtypeStruct((_N, _W), F32)] * 2,
)

_k3 = pl.pallas_call(
    _k3_body,
    grid=(_NB,),
    in_specs=[
        pl.BlockSpec((_RB, _W), lambda i: (i, 0)),
        pl.BlockSpec((_NC, _RB, _W), lambda i: (0, i, 0)),
        pl.BlockSpec((20, 10), lambda i: (0, 0)),
    ],
    out_specs=[
        pl.BlockSpec((_RB, _W), lambda i: (i, 0)),
        pl.BlockSpec((_RB, _W), lambda i: (i, 0)),
    ],
    out_shape=[jax.ShapeDtypeStruct((_N, _W), F32)] * 2,
)

_k5 = pl.pallas_call(
    _k5_body,
    grid=(_NB,),
    in_specs=[
        pl.BlockSpec((_RB, _W), lambda i: (i, 0)),
        pl.BlockSpec((_NC, _RB, _W), lambda i: (0, i, 0)),
        pl.BlockSpec((1, 1, _RB), lambda i: (i, 0, 0)),
        pl.BlockSpec((10, 1), lambda i: (0, 0)),
    ],
    out_specs=pl.BlockSpec((_G, 1), lambda i: (0, 0)),
    out_shape=jax.ShapeDtypeStruct((_G, 1), F32),
    scratch_shapes=[pltpu.VMEM((_G, _W), F32)],
)


def kernel(x, edge_index, batch, W1, W2, Wfc):
    edges = edge_index.reshape(2, _NCH, _CH)
    zeros = jnp.zeros((_N, _W), F32)
    batch3 = batch.reshape(_NB, 1, _RB)

    segsum = _make_segsum()
    self1, tab1 = _k1(x, W1)
    p1 = segsum(tab1, edges, zeros)
    self2, tab2 = _k3(self1, p1, W2)
    p2 = segsum(tab2, edges, zeros)
    return _k5(self2, p2, batch3, Wfc)
